# baseline (speedup 1.0000x reference)
import functools

import jax
import jax.numpy as jnp
from jax.experimental import pallas as pl
from jax.experimental.pallas import tpu as pltpu

B, S, D = 4, 2048, 1024
ITERS = 4
NUM_LAYERS = 4
LN_EPS = 1e-5
T = B * S
TM = 1024
_NC = 8


def _fused_kernel(h_ref, m_ref, g_ref, ones_ref, oh_ref, acd_ref, wl_ref,
                  out_ref, probs_ref):
    hb = h_ref[...]
    m = m_ref[...]
    dp = float(D + ITERS)
    oh = oh_ref[...]
    oh_s1 = jnp.sum(oh)
    oh_s2 = jnp.sum(oh * oh)
    r = jnp.dot(hb, g_ref[...], preferred_element_type=jnp.float32)
    r2 = jnp.dot(hb * hb, ones_ref[...], preferred_element_type=jnp.float32)
    s1 = m * r[:, NUM_LAYERS:NUM_LAYERS + 1]
    s2 = (m * m) * r2[:, 0:1]
    mean = (s1 + oh_s1) / dp
    var = (s2 + oh_s2) / dp - mean * mean
    inv = jax.lax.rsqrt(var + LN_EPS)
    a_row = acd_ref[0:1, :NUM_LAYERS]
    c_row = acd_ref[1:2, :NUM_LAYERS]
    d_row = acd_ref[2:3, :NUM_LAYERS]
    logits = inv * (m * r[:, :NUM_LAYERS] + d_row - mean * a_row) + c_row
    lmax = jnp.max(logits, axis=1, keepdims=True)
    e = jnp.exp(logits - lmax)
    p = e / jnp.sum(e, axis=1, keepdims=True)
    probs_ref[...] = p
    w = p * m
    acc = None
    for l in range(NUM_LAYERS):
        y = jnp.dot(hb, wl_ref[l], preferred_element_type=jnp.float32)
        wy = w[:, l:l + 1] * y
        acc = wy if acc is None else acc + wy
    out_ref[...] = acc


@functools.partial(jax.jit, static_argnames=())
def kernel(hidden_states, attention_mask, ln_scale, ln_bias, W_router,
           W_layers, iteration):
    hb2d = hidden_states.reshape(T, D).astype(jnp.bfloat16)
    mask2 = attention_mask.reshape(T, 1).astype(jnp.float32)
    oh = jax.nn.one_hot(iteration, ITERS, dtype=jnp.float32)
    wg = (ln_scale[:D][:, None] * W_router[:, :D].T)
    g = jnp.concatenate(
        [wg, jnp.ones((D, 1), jnp.float32),
         jnp.zeros((D, _NC - NUM_LAYERS - 1), jnp.float32)],
        axis=1).astype(jnp.bfloat16)
    ones_col = jnp.concatenate(
        [jnp.ones((D, 1), jnp.float32),
         jnp.zeros((D, _NC - 1), jnp.float32)],
        axis=1).astype(jnp.bfloat16)
    a = (ln_scale[None, :] * W_router).sum(axis=1)
    c = W_router @ ln_bias
    dt = (W_router[:, D:] * (ln_scale[D:] * oh)[None, :]).sum(axis=1)
    acd = jnp.stack([a, c, dt], axis=0)
    oh_row = oh.reshape(1, ITERS)
    wl_bf = W_layers.astype(jnp.bfloat16)

    full = lambda shp: pl.BlockSpec(shp, lambda i: (0,) * len(shp))
    grid = (T // TM,)
    out, probs = pl.pallas_call(
        _fused_kernel,
        grid=grid,
        in_specs=[
            pl.BlockSpec((TM, D), lambda i: (i, 0)),
            pl.BlockSpec((TM, 1), lambda i: (i, 0)),
            full((D, _NC)), full((D, _NC)),
            full((1, ITERS)), full((3, NUM_LAYERS)),
            full((NUM_LAYERS, D, D)),
        ],
        out_specs=[
            pl.BlockSpec((TM, D), lambda i: (i, 0)),
            pl.BlockSpec((TM, NUM_LAYERS), lambda i: (i, 0)),
        ],
        out_shape=[
            jax.ShapeDtypeStruct((T, D), jnp.float32),
            jax.ShapeDtypeStruct((T, NUM_LAYERS), jnp.float32),
        ],
        compiler_params=pltpu.CompilerParams(
            dimension_semantics=("parallel",),
        ),
    )(hb2d, mask2, g, ones_col, oh_row, acd, wl_bf)
    return out.reshape(B, S, D), probs.reshape(B, S, NUM_LAYERS)


# speedup vs baseline: 1.1519x; 1.1519x over previous
import functools

import jax
import jax.numpy as jnp
from jax.experimental import pallas as pl
from jax.experimental.pallas import tpu as pltpu

B, S, D = 4, 2048, 1024
ITERS = 4
NUM_LAYERS = 4
LN_EPS = 1e-5
T = B * S
TM = 1024
_NC = 8


def _fused_kernel(h_ref, m_ref, g_ref, ones_ref, oh_ref, acd_ref, wl_ref,
                  out_ref, probs_ref):
    hb = h_ref[...].astype(jnp.bfloat16)
    m = m_ref[...]
    dp = float(D + ITERS)
    oh = oh_ref[...]
    oh_s1 = jnp.sum(oh)
    oh_s2 = jnp.sum(oh * oh)
    r = jnp.dot(hb, g_ref[...], preferred_element_type=jnp.float32)
    r2 = jnp.dot(hb * hb, ones_ref[...], preferred_element_type=jnp.float32)
    s1 = m * r[:, NUM_LAYERS:NUM_LAYERS + 1]
    s2 = (m * m) * r2[:, 0:1]
    mean = (s1 + oh_s1) / dp
    var = (s2 + oh_s2) / dp - mean * mean
    inv = jax.lax.rsqrt(var + LN_EPS)
    a_row = acd_ref[0:1, :NUM_LAYERS]
    c_row = acd_ref[1:2, :NUM_LAYERS]
    d_row = acd_ref[2:3, :NUM_LAYERS]
    logits = inv * (m * r[:, :NUM_LAYERS] + d_row - mean * a_row) + c_row
    lmax = jnp.max(logits, axis=1, keepdims=True)
    e = jnp.exp(logits - lmax)
    p = e / jnp.sum(e, axis=1, keepdims=True)
    probs_ref[...] = p
    w = p * m
    acc = None
    for l in range(NUM_LAYERS):
        y = jnp.dot(hb, wl_ref[l], preferred_element_type=jnp.float32)
        wy = w[:, l:l + 1] * y
        acc = wy if acc is None else acc + wy
    out_ref[...] = acc


@functools.partial(jax.jit, static_argnames=())
def kernel(hidden_states, attention_mask, ln_scale, ln_bias, W_router,
           W_layers, iteration):
    hb2d = hidden_states.reshape(T, D)
    mask2 = attention_mask.reshape(T, 1).astype(jnp.float32)
    oh = jax.nn.one_hot(iteration, ITERS, dtype=jnp.float32)
    wg = (ln_scale[:D][:, None] * W_router[:, :D].T)
    g = jnp.concatenate(
        [wg, jnp.ones((D, 1), jnp.float32),
         jnp.zeros((D, _NC - NUM_LAYERS - 1), jnp.float32)],
        axis=1).astype(jnp.bfloat16)
    ones_col = jnp.concatenate(
        [jnp.ones((D, 1), jnp.float32),
         jnp.zeros((D, _NC - 1), jnp.float32)],
        axis=1).astype(jnp.bfloat16)
    a = (ln_scale[None, :] * W_router).sum(axis=1)
    c = W_router @ ln_bias
    dt = (W_router[:, D:] * (ln_scale[D:] * oh)[None, :]).sum(axis=1)
    acd = jnp.stack([a, c, dt], axis=0)
    oh_row = oh.reshape(1, ITERS)
    wl_bf = W_layers.astype(jnp.bfloat16)

    full = lambda shp: pl.BlockSpec(shp, lambda i: (0,) * len(shp))
    grid = (T // TM,)
    out, probs = pl.pallas_call(
        _fused_kernel,
        grid=grid,
        in_specs=[
            pl.BlockSpec((TM, D), lambda i: (i, 0)),
            pl.BlockSpec((TM, 1), lambda i: (i, 0)),
            full((D, _NC)), full((D, _NC)),
            full((1, ITERS)), full((3, NUM_LAYERS)),
            full((NUM_LAYERS, D, D)),
        ],
        out_specs=[
            pl.BlockSpec((TM, D), lambda i: (i, 0)),
            pl.BlockSpec((TM, NUM_LAYERS), lambda i: (i, 0)),
        ],
        out_shape=[
            jax.ShapeDtypeStruct((T, D), jnp.float32),
            jax.ShapeDtypeStruct((T, NUM_LAYERS), jnp.float32),
        ],
        compiler_params=pltpu.CompilerParams(
            dimension_semantics=("parallel",),
        ),
    )(hb2d, mask2, g, ones_col, oh_row, acd, wl_bf)
    return out.reshape(B, S, D), probs.reshape(B, S, NUM_LAYERS)
